# baseline (speedup 1.0000x reference)
import jax
import jax.numpy as jnp
from jax.experimental import pallas as pl
from jax.experimental.pallas import tpu as pltpu

B = 1024
NUM_FEATURES = 128
NUM_CLASSES = 100000
NBLK = 2048


def _mm_kernel(x_ref, m_ref, o_ref):
    o_ref[...] = jax.lax.dot_general(
        x_ref[...], m_ref[...],
        dimension_numbers=(((1,), (1,)), ((), ())),
        preferred_element_type=jnp.float32,
    )


def kernel(inputs, targets, epoch, mem):
    del targets, epoch
    nblocks = pl.cdiv(NUM_CLASSES, NBLK)
    return pl.pallas_call(
        _mm_kernel,
        grid=(nblocks,),
        in_specs=[
            pl.BlockSpec((B, NUM_FEATURES), lambda j: (0, 0)),
            pl.BlockSpec((NBLK, NUM_FEATURES), lambda j: (j, 0)),
        ],
        out_specs=pl.BlockSpec((B, NBLK), lambda j: (0, j)),
        out_shape=jax.ShapeDtypeStruct((B, NUM_CLASSES), jnp.float32),
        compiler_params=pltpu.CompilerParams(
            dimension_semantics=("parallel",),
        ),
    )(inputs, mem)


# speedup vs baseline: 1.0063x; 1.0063x over previous
import jax
import jax.numpy as jnp
from jax.experimental import pallas as pl
from jax.experimental.pallas import tpu as pltpu

B = 1024
NUM_FEATURES = 128
NUM_CLASSES = 100000
NBLK = 4096


def _mm_kernel(x_ref, m_ref, o_ref):
    o_ref[...] = jax.lax.dot_general(
        x_ref[...], m_ref[...],
        dimension_numbers=(((1,), (1,)), ((), ())),
        preferred_element_type=jnp.float32,
    )


def kernel(inputs, targets, epoch, mem):
    del targets, epoch
    nblocks = pl.cdiv(NUM_CLASSES, NBLK)
    return pl.pallas_call(
        _mm_kernel,
        grid=(nblocks,),
        in_specs=[
            pl.BlockSpec((B, NUM_FEATURES), lambda j: (0, 0)),
            pl.BlockSpec((NBLK, NUM_FEATURES), lambda j: (j, 0)),
        ],
        out_specs=pl.BlockSpec((B, NBLK), lambda j: (0, j)),
        out_shape=jax.ShapeDtypeStruct((B, NUM_CLASSES), jnp.float32),
        compiler_params=pltpu.CompilerParams(
            dimension_semantics=("parallel",),
        ),
    )(inputs, mem)


# speedup vs baseline: 1.1007x; 1.0937x over previous
import jax
import jax.numpy as jnp
from jax.experimental import pallas as pl
from jax.experimental.pallas import tpu as pltpu

B = 1024
NUM_FEATURES = 128
NUM_CLASSES = 100000
NBLK = 1024
NBUF = 8
GRID = NUM_CLASSES // NBLK + 1
ALIGNED = NUM_CLASSES // 128 * 128
TAILW = ALIGNED - (GRID - 1) * NBLK
RAG = NUM_CLASSES - ALIGNED


def _mm_kernel(x_ref, m_ref, o_hbm, rag_ref, scratch, tail, sems, tail_sem):
    j = pl.program_id(0)
    buf = jax.lax.rem(j, NBUF)

    @pl.when(j >= NBUF)
    def _wait_reuse():
        pltpu.make_async_copy(
            scratch.at[buf], o_hbm.at[:, pl.ds(0, NBLK)], sems.at[buf]
        ).wait()

    val = jax.lax.dot_general(
        x_ref[...], m_ref[...],
        dimension_numbers=(((1,), (1,)), ((), ())),
        preferred_element_type=jnp.float32,
    )

    @pl.when(j < GRID - 1)
    def _copy_full():
        scratch[buf] = val
        pltpu.make_async_copy(
            scratch.at[buf], o_hbm.at[:, pl.ds(j * NBLK, NBLK)], sems.at[buf]
        ).start()

    @pl.when(j == GRID - 1)
    def _copy_tail_and_drain():
        tail[...] = val[:, :TAILW]
        rag_ref[...] = val[:, TAILW:TAILW + RAG]
        pltpu.make_async_copy(
            tail, o_hbm.at[:, pl.ds((GRID - 1) * NBLK, TAILW)], tail_sem
        ).start()
        for k in range(GRID - NBUF, GRID - 1):
            b = k % NBUF
            pltpu.make_async_copy(
                scratch.at[b], o_hbm.at[:, pl.ds(0, NBLK)], sems.at[b]
            ).wait()
        pltpu.make_async_copy(
            tail, o_hbm.at[:, pl.ds(0, TAILW)], tail_sem
        ).wait()


def kernel(inputs, targets, epoch, mem):
    del targets, epoch
    main, rag = pl.pallas_call(
        _mm_kernel,
        grid=(GRID,),
        in_specs=[
            pl.BlockSpec((B, NUM_FEATURES), lambda j: (0, 0)),
            pl.BlockSpec((NBLK, NUM_FEATURES), lambda j: (j, 0)),
        ],
        out_specs=[
            pl.BlockSpec(memory_space=pltpu.MemorySpace.HBM),
            pl.BlockSpec((B, RAG), lambda j: (0, 0)),
        ],
        out_shape=[
            jax.ShapeDtypeStruct((B, NUM_CLASSES), jnp.float32),
            jax.ShapeDtypeStruct((B, RAG), jnp.float32),
        ],
        scratch_shapes=[
            pltpu.VMEM((NBUF, B, NBLK), jnp.float32),
            pltpu.VMEM((B, TAILW), jnp.float32),
            pltpu.SemaphoreType.DMA((NBUF,)),
            pltpu.SemaphoreType.DMA,
        ],
        compiler_params=pltpu.CompilerParams(
            dimension_semantics=("arbitrary",),
        ),
    )(inputs, mem)
    return jax.lax.dynamic_update_slice(main, rag, (0, ALIGNED))
